# baseline (speedup 1.0000x reference)
import functools

import jax
import jax.numpy as jnp
from jax import lax
from jax.experimental import pallas as pl
from jax.experimental.pallas import tpu as pltpu
from jax.experimental.pallas import tpu_sc as plsc

_VOCAB = 6
_EPS = 1e-5
_SC_SEQ = 48


def _normed_rows(tok_ref, pos, gamma, beta, v):
    row = tok_ref[pl.ds(v, 1), :] + pos
    mean = jnp.mean(row, axis=-1, keepdims=True)
    cent = row - mean
    var = jnp.mean(cent * cent, axis=-1, keepdims=True)
    normed = cent * jax.lax.rsqrt(var + _EPS)
    return normed * gamma + beta


def _table_body(tok_ref, pos_ref, gamma_ref, beta_ref, out_ref):
    seq, d = pos_ref.shape
    pos = pos_ref[...]
    gamma = gamma_ref[...]
    beta = beta_ref[...]
    for v in range(_VOCAB):
        out_ref[pl.ds(v * seq, seq), :] = _normed_rows(tok_ref, pos, gamma, beta, v)


def _tail_body(x_ref, tok_ref, pos_ref, gamma_ref, beta_ref, _, out_ref, vmem, sem):
    d = pos_ref.shape[-1]
    b_blk = x_ref.shape[0]
    pos = pos_ref[...]
    gamma = gamma_ref[...]
    beta = beta_ref[...]
    xb = x_ref[...]
    acc = None
    for v in range(_VOCAB):
        normed = _normed_rows(tok_ref, pos, gamma, beta, v)
        if acc is None:
            acc = jnp.broadcast_to(normed[None], (b_blk, 2, d))
        else:
            acc = jnp.where(xb == v, normed[None], acc)
    vmem[...] = acc
    i = pl.program_id(0)
    pltpu.make_async_copy(
        vmem,
        out_ref.at[pl.ds(i * b_blk, b_blk), pl.ds(_SC_SEQ, 2), :],
        sem,
    ).start()
    pltpu.make_async_copy(
        vmem,
        out_ref.at[pl.ds(i * b_blk, b_blk), pl.ds(_SC_SEQ, 2), :],
        sem,
    ).wait()


@jax.jit
def _run(x, tok_table, pos_table, gamma, beta):
    batch, seq = x.shape
    d = tok_table.shape[-1]
    gamma2 = gamma.reshape(1, d)
    beta2 = beta.reshape(1, d)

    table = pl.pallas_call(
        _table_body,
        out_shape=jax.ShapeDtypeStruct((_VOCAB * seq, d), jnp.float32),
    )(tok_table, pos_table, gamma2, beta2)

    cidx = x[:, :_SC_SEQ] * seq + jnp.arange(_SC_SEQ, dtype=jnp.int32)[None, :]

    info = plsc.get_sparse_core_info()
    nw = info.num_cores * info.num_subcores
    n_per = batch // nw
    mesh = plsc.VectorSubcoreMesh(core_axis_name="c", subcore_axis_name="s")

    @functools.partial(
        pl.kernel,
        out_type=jax.ShapeDtypeStruct((batch, seq, d), jnp.float32),
        mesh=mesh,
        scratch_types=[
            pltpu.VMEM((n_per, _SC_SEQ), jnp.int32),
            pltpu.VMEM((_SC_SEQ, d), jnp.float32),
            pltpu.SemaphoreType.DMA,
        ],
    )
    def sc_lookup(table_hbm, cidx_hbm, out_hbm, idx_v, rows_v, sem):
        wid = lax.axis_index("s") * info.num_cores + lax.axis_index("c")
        base = wid * n_per
        pltpu.sync_copy(cidx_hbm.at[pl.ds(base, n_per)], idx_v)

        def body(i, carry):
            pltpu.async_copy(table_hbm.at[idx_v.at[i]], rows_v, sem).wait()
            pltpu.sync_copy(rows_v, out_hbm.at[base + i, pl.ds(0, _SC_SEQ)])
            return carry

        lax.fori_loop(0, n_per, body, 0)

    sc_out = sc_lookup(table, cidx)

    b_blk = 256
    tail = pl.pallas_call(
        _tail_body,
        grid=(batch // b_blk,),
        in_specs=[
            pl.BlockSpec((b_blk, 2, 1), lambda i: (i, 0, 0)),
            pl.BlockSpec((_VOCAB, d), lambda i: (0, 0)),
            pl.BlockSpec((2, d), lambda i: (0, 0)),
            pl.BlockSpec((1, d), lambda i: (0, 0)),
            pl.BlockSpec((1, d), lambda i: (0, 0)),
            pl.BlockSpec(memory_space=pltpu.MemorySpace.HBM),
        ],
        out_specs=pl.BlockSpec(memory_space=pl.ANY),
        out_shape=jax.ShapeDtypeStruct((batch, seq, d), jnp.float32),
        input_output_aliases={5: 0},
        scratch_shapes=[
            pltpu.VMEM((b_blk, 2, d), jnp.float32),
            pltpu.SemaphoreType.DMA,
        ],
        compiler_params=pltpu.CompilerParams(
            dimension_semantics=("arbitrary",),
        ),
    )(x[:, _SC_SEQ:, None], tok_table, pos_table[_SC_SEQ:], gamma2, beta2, sc_out)
    return tail


def kernel(x, tok_table, pos_table, gamma, beta):
    return _run(x, tok_table, pos_table, gamma, beta)


# speedup vs baseline: 1.0324x; 1.0324x over previous
import functools

import jax
import jax.numpy as jnp
from jax import lax
from jax.experimental import pallas as pl
from jax.experimental.pallas import tpu as pltpu
from jax.experimental.pallas import tpu_sc as plsc

_VOCAB = 6
_EPS = 1e-5
_SC_SEQ = 48


def _normed_rows(tok_ref, pos, gamma, beta, v):
    row = tok_ref[pl.ds(v, 1), :] + pos
    mean = jnp.mean(row, axis=-1, keepdims=True)
    cent = row - mean
    var = jnp.mean(cent * cent, axis=-1, keepdims=True)
    normed = cent * jax.lax.rsqrt(var + _EPS)
    return normed * gamma + beta


def _table_body(tok_ref, pos_ref, gamma_ref, beta_ref, out_ref):
    seq, d = pos_ref.shape
    pos = pos_ref[...]
    gamma = gamma_ref[...]
    beta = beta_ref[...]
    for v in range(_VOCAB):
        out_ref[pl.ds(v * seq, seq), :] = _normed_rows(tok_ref, pos, gamma, beta, v)


def _tail_body(x_ref, tok_ref, pos_ref, gamma_ref, beta_ref, _, out_ref, vmem, sem):
    d = pos_ref.shape[-1]
    b_blk = x_ref.shape[0]
    pos = pos_ref[...]
    gamma = gamma_ref[...]
    beta = beta_ref[...]
    xb = x_ref[...]
    acc = None
    for v in range(_VOCAB):
        normed = _normed_rows(tok_ref, pos, gamma, beta, v)
        if acc is None:
            acc = jnp.broadcast_to(normed[None], (b_blk, 2, d))
        else:
            acc = jnp.where(xb == v, normed[None], acc)
    vmem[...] = acc
    i = pl.program_id(0)
    pltpu.make_async_copy(
        vmem,
        out_ref.at[pl.ds(i * b_blk, b_blk), pl.ds(_SC_SEQ, 2), :],
        sem,
    ).start()
    pltpu.make_async_copy(
        vmem,
        out_ref.at[pl.ds(i * b_blk, b_blk), pl.ds(_SC_SEQ, 2), :],
        sem,
    ).wait()


@jax.jit
def _run(x, tok_table, pos_table, gamma, beta):
    batch, seq = x.shape
    d = tok_table.shape[-1]
    gamma2 = gamma.reshape(1, d)
    beta2 = beta.reshape(1, d)

    table = pl.pallas_call(
        _table_body,
        out_shape=jax.ShapeDtypeStruct((_VOCAB * seq, d), jnp.float32),
    )(tok_table, pos_table, gamma2, beta2)

    cidx = x[:, :_SC_SEQ] * seq + jnp.arange(_SC_SEQ, dtype=jnp.int32)[None, :]
    cidx = cidx.reshape(batch // 2, 2 * _SC_SEQ)

    info = plsc.get_sparse_core_info()
    nw = info.num_cores * info.num_subcores
    units_per = (batch // 2) // nw
    mesh = plsc.VectorSubcoreMesh(core_axis_name="c", subcore_axis_name="s")

    @functools.partial(
        pl.kernel,
        out_type=jax.ShapeDtypeStruct((batch, seq, d), jnp.float32),
        mesh=mesh,
        scratch_types=[
            pltpu.VMEM((units_per, 2 * _SC_SEQ), jnp.int32),
            pltpu.VMEM((2 * _SC_SEQ, d), jnp.float32),
            pltpu.SemaphoreType.DMA,
        ],
    )
    def sc_lookup(table_hbm, cidx_hbm, out_hbm, idx_v, rows_v, sem):
        wid = lax.axis_index("s") * info.num_cores + lax.axis_index("c")
        base = wid * units_per
        pltpu.sync_copy(cidx_hbm.at[pl.ds(base, units_per)], idx_v)

        def body(i, carry):
            p0 = (base + i) * 2
            pltpu.async_copy(table_hbm.at[idx_v.at[i]], rows_v, sem).wait()
            pltpu.sync_copy(
                rows_v.at[pl.ds(0, _SC_SEQ)],
                out_hbm.at[p0, pl.ds(0, _SC_SEQ)],
            )
            pltpu.sync_copy(
                rows_v.at[pl.ds(_SC_SEQ, _SC_SEQ)],
                out_hbm.at[p0 + 1, pl.ds(0, _SC_SEQ)],
            )
            return carry

        lax.fori_loop(0, units_per, body, 0)

    sc_out = sc_lookup(table, cidx)

    b_blk = 256
    tail = pl.pallas_call(
        _tail_body,
        grid=(batch // b_blk,),
        in_specs=[
            pl.BlockSpec((b_blk, 2, 1), lambda i: (i, 0, 0)),
            pl.BlockSpec((_VOCAB, d), lambda i: (0, 0)),
            pl.BlockSpec((2, d), lambda i: (0, 0)),
            pl.BlockSpec((1, d), lambda i: (0, 0)),
            pl.BlockSpec((1, d), lambda i: (0, 0)),
            pl.BlockSpec(memory_space=pltpu.MemorySpace.HBM),
        ],
        out_specs=pl.BlockSpec(memory_space=pl.ANY),
        out_shape=jax.ShapeDtypeStruct((batch, seq, d), jnp.float32),
        input_output_aliases={5: 0},
        scratch_shapes=[
            pltpu.VMEM((b_blk, 2, d), jnp.float32),
            pltpu.SemaphoreType.DMA,
        ],
        compiler_params=pltpu.CompilerParams(
            dimension_semantics=("arbitrary",),
        ),
    )(x[:, _SC_SEQ:, None], tok_table, pos_table[_SC_SEQ:], gamma2, beta2, sc_out)
    return tail


def kernel(x, tok_table, pos_table, gamma, beta):
    return _run(x, tok_table, pos_table, gamma, beta)
